# baseline (speedup 1.0000x reference)
import functools

import jax
import jax.numpy as jnp
from jax import lax
from jax.experimental import pallas as pl
from jax.experimental.pallas import tpu as pltpu
from jax.experimental.pallas import tpu_sc as plsc

N = 10000
E = 160000
NK = 64
NSC = 2
NT = 16
EPC = E // NSC
NPT = N // NT
BE = 32
NBLK = EPC // (NT * BE)
NBLK_REM = EPC // BE - NBLK * NT

_f32 = jnp.float32
_i32 = jnp.int32


_PBE = 1280


def _prep_body(p_ref, s_ref, w_ref, g_ref):
    v = p_ref[...] * 3.0
    bot = jnp.clip(jnp.floor(v), 0.0, 2.0)
    frac = v - bot
    boti = bot.astype(_i32)
    src = s_ref[...]
    ws, gs = [], []
    for s in range(8):
        w = jnp.ones((1, _PBE), _f32)
        idx = jnp.zeros((1, _PBE), _i32)
        for d in range(3):
            bit = (s >> d) & 1
            fd = frac[d:d + 1, :]
            w = w * (fd if bit else (1.0 - fd))
            idx = idx + (boti[d:d + 1, :] + bit) * (4 ** d)
        ws.append(w)
        gs.append(src * NK + idx)
    w_ref[...] = jnp.concatenate(ws, axis=0)
    g_ref[...] = jnp.concatenate(gs, axis=0)


def _prep(pseudoT, srcT):
    grid = E // _PBE
    return pl.pallas_call(
        _prep_body,
        grid=(grid,),
        in_specs=[
            pl.BlockSpec((3, _PBE), lambda i: (0, i)),
            pl.BlockSpec((1, _PBE), lambda i: (0, i)),
        ],
        out_specs=[
            pl.BlockSpec((8, _PBE), lambda i: (0, i)),
            pl.BlockSpec((8, _PBE), lambda i: (0, i)),
        ],
        out_shape=[
            jax.ShapeDtypeStruct((8, E), _f32),
            jax.ShapeDtypeStruct((8, E), _i32),
        ],
    )(pseudoT, srcT)


_Y1BN = 200


def _y1_body(x_ref, w_ref, o_ref):
    xb = x_ref[...]
    w = w_ref[...]
    t = xb[:, :, None] * w[None, :, :]
    col = (lax.broadcasted_iota(_i32, (1, 1, 128), 2) == NK).astype(_f32)
    o_ref[...] = (t + col).reshape(_Y1BN * NK, 128)


def _y1build(x, w1pad):
    grid = N // _Y1BN
    return pl.pallas_call(
        _y1_body,
        grid=(grid,),
        in_specs=[
            pl.BlockSpec((_Y1BN, 1), lambda i: (i, 0)),
            pl.BlockSpec((NK, 128), lambda i: (0, 0)),
        ],
        out_specs=pl.BlockSpec((_Y1BN * NK, 128), lambda i: (i, 0)),
        out_shape=jax.ShapeDtypeStruct((N * NK, 128), _f32),
    )(x, w1pad)


def _mm_body(a_ref, b_ref, o_ref):
    o_ref[...] = jnp.dot(a_ref[...], b_ref[...], preferred_element_type=_f32)


def _matmul(a, b, bm=1000, bn=512):
    m, k = a.shape
    k2, n = b.shape
    assert k == k2 and m % bm == 0
    bn = min(bn, n)
    assert n % bn == 0
    return pl.pallas_call(
        _mm_body,
        grid=(m // bm, n // bn),
        in_specs=[
            pl.BlockSpec((bm, k), lambda i, j: (i, 0)),
            pl.BlockSpec((k, bn), lambda i, j: (0, j)),
        ],
        out_specs=pl.BlockSpec((bm, bn), lambda i, j: (i, j)),
        out_shape=jax.ShapeDtypeStruct((m, n), _f32),
    )(a, b)


@functools.lru_cache(maxsize=None)
def _make_gcs(D, interpret=False):
    NCH = N // 16
    mesh = plsc.VectorSubcoreMesh(core_axis_name="c", subcore_axis_name="s")

    @functools.partial(
        pl.kernel,
        out_type=jax.ShapeDtypeStruct((NSC, N, D), _f32),
        mesh=mesh,
        scratch_types=[
            pltpu.VMEM((8, BE), _i32),
            pltpu.VMEM((8, BE + 16), _f32),
            pltpu.VMEM((BE,), _i32),
            pltpu.VMEM((8, BE, D), _f32),
            pltpu.VMEM((BE, D), _f32),
            pltpu.VMEM((16, D), _f32),
            pltpu.VMEM_SHARED((N, D), _f32),
            pltpu.SemaphoreType.DMA,
        ],
        interpret=interpret,
    )
    def gcs(tab_hbm, gidx_hbm, wgt_hbm, dst_hbm, out_hbm,
            idx_v, w_v, dst_v, rows_v, msg_v, zero_v, acc_sh, sem):
        c = lax.axis_index("c")
        s = lax.axis_index("s")

        def zrow(r, _):
            for j in range(D // 16):
                zero_v[r, pl.ds(j * 16, 16)] = jnp.zeros((16,), _f32)
            return 0
        lax.fori_loop(0, 16, zrow, 0)
        nchunk = (NCH - s + NT - 1) // NT

        def zchunk(t, _):
            q = s + t * NT
            pltpu.sync_copy(zero_v, acc_sh.at[pl.ds(q * 16, 16)])
            return 0
        lax.fori_loop(0, nchunk, zchunk, 0)
        plsc.subcore_barrier()

        extra = (s < NBLK_REM).astype(_i32)
        base = c * EPC + (s * NBLK + jnp.minimum(s, NBLK_REM)) * BE
        nblk = NBLK + extra

        def blk(b, _):
            e0 = base + b * BE
            for k in range(8):
                pltpu.sync_copy(gidx_hbm.at[pl.ds(k * E + e0, BE)],
                                idx_v.at[k])
                pltpu.sync_copy(wgt_hbm.at[pl.ds(k * E + e0, BE)],
                                w_v.at[k, pl.ds(0, BE)])
            pltpu.sync_copy(dst_hbm.at[pl.ds(e0, BE)], dst_v)
            cps = [pltpu.async_copy(tab_hbm.at[idx_v.at[k]], rows_v.at[k], sem)
                   for k in range(8)]
            for cp in cps:
                cp.wait()

            def edge(e, _):
                ws = [w_v[k, pl.ds(e, 16)][0] for k in range(8)]
                for j in range(D // 16):
                    sl = pl.ds(j * 16, 16)
                    acc = ws[0] * rows_v[0, e, sl]
                    for k in range(1, 8):
                        acc = acc + ws[k] * rows_v[k, e, sl]
                    msg_v[e, sl] = acc
                return 0
            lax.fori_loop(0, BE, edge, 0)
            pltpu.sync_copy(msg_v, acc_sh.at[dst_v], add=True)
            return 0
        lax.fori_loop(0, nblk, blk, 0)

        plsc.subcore_barrier()

        def wchunk(t, _):
            q = s + t * NT
            pltpu.sync_copy(acc_sh.at[pl.ds(q * 16, 16)],
                            out_hbm.at[c, pl.ds(q * 16, 16)])
            return 0
        lax.fori_loop(0, nchunk, wchunk, 0)

    return gcs


def _post_body(co, first, last, parts_ref, x_ref, root_ref, bias_ref,
               g_ref, b_ref, deg_ref, o_ref, degout_ref=None):
    psum = parts_ref[0] + parts_ref[1]
    agg = psum[:, :co]
    if first:
        deg = jnp.clip(psum[:, NK:NK + 1], 1.0, None)
        degout_ref[...] = deg
    else:
        deg = deg_ref[...]
    h = agg / deg + jnp.dot(x_ref[...], root_ref[...],
                            preferred_element_type=_f32) + bias_ref[...]
    h = jnp.where(h > 0, h, jnp.exp(h) - 1.0)
    mu = jnp.mean(h, axis=0, keepdims=True)
    var = jnp.mean((h - mu) ** 2, axis=0, keepdims=True)
    h = (h - mu) / jnp.sqrt(var + 1e-5) * g_ref[...] + b_ref[...]
    if last:
        shifted = h - jnp.max(h, axis=1, keepdims=True)
        h = shifted - jnp.log(jnp.sum(jnp.exp(shifted), axis=1, keepdims=True))
    o_ref[...] = h


def _post(parts, x, root, bias, g, b, deg, first=False, last=False):
    co = root.shape[1]
    D = parts.shape[2]
    outs = [jax.ShapeDtypeStruct((N, co), _f32)]
    if first:
        outs.append(jax.ShapeDtypeStruct((N, 1), _f32))
    res = pl.pallas_call(
        functools.partial(_post_body, co, first, last),
        out_shape=outs,
    )(parts, x, root, bias.reshape(1, co), g.reshape(1, co),
      b.reshape(1, co), deg)
    return res if first else res[0]


def kernel(x, edge_index, edge_attr, W1, root1, bias1, g1, b1,
           W2, root2, bias2, g2, b2, W3, root3, bias3, g3, b3,
           W4, root4, bias4, g4, b4):
    src = edge_index[0].astype(_i32)
    dst = edge_index[1].astype(_i32)
    pseudoT = edge_attr.T
    wgt8, gidx8 = _prep(pseudoT, src.reshape(1, E))
    wgt8 = wgt8.reshape(8 * E)
    gidx8 = gidx8.reshape(8 * E)

    w1pad = jnp.pad(W1.reshape(NK, 64), ((0, 0), (0, 64)))
    tab1 = _y1build(x, w1pad)
    parts1 = _make_gcs(128)(tab1, gidx8, wgt8, dst)
    h1, deg = _post(parts1, x, root1, bias1, g1, b1, jnp.zeros((N, 1), _f32),
                    first=True)

    w2cat = W2.transpose(1, 0, 2).reshape(64, NK * 128)
    tab2 = _matmul(h1, w2cat).reshape(N * NK, 128)
    parts2 = _make_gcs(128)(tab2, gidx8, wgt8, dst)
    h2 = _post(parts2, h1, root2, bias2, g2, b2, deg)

    halves = []
    for p in range(2):
        w3p = W3[:, :, p * 128:(p + 1) * 128].transpose(1, 0, 2)
        tabp = _matmul(h2, w3p.reshape(128, NK * 128)).reshape(N * NK, 128)
        partsp = _make_gcs(128)(tabp, gidx8, wgt8, dst)
        halves.append(_post(partsp, h2, root3[:, p * 128:(p + 1) * 128],
                            bias3[p * 128:(p + 1) * 128],
                            g3[p * 128:(p + 1) * 128],
                            b3[p * 128:(p + 1) * 128], deg))
    h3 = jnp.concatenate(halves, axis=1)

    y4 = _matmul(h3, W4[:, :, 0].T)
    tab4 = jnp.pad(y4.reshape(N * NK, 1), ((0, 0), (0, 127)))
    parts4 = _make_gcs(128)(tab4, gidx8, wgt8, dst)
    out = _post(parts4, h3, root4, bias4, g4, b4, deg, last=True)
    return out


# speedup vs baseline: 2.1242x; 2.1242x over previous
import functools

import jax
import jax.numpy as jnp
from jax import lax
from jax.experimental import pallas as pl
from jax.experimental.pallas import tpu as pltpu
from jax.experimental.pallas import tpu_sc as plsc

N = 10000
E = 160000
NK = 64
NSC = 2
NT = 16
EPC = E // NSC
NPT = N // NT
BE = 16
BPT = 313
EP = NSC * NT * BPT * BE

_f32 = jnp.float32
_i32 = jnp.int32


_PBE = 1280


def _prep_body(p_ref, s_ref, w_ref, g_ref):
    v = p_ref[...] * 3.0
    bot = jnp.clip(jnp.floor(v), 0.0, 2.0)
    frac = v - bot
    boti = bot.astype(_i32)
    src = s_ref[...]
    ws, gs = [], []
    for s in range(8):
        w = jnp.ones((1, _PBE), _f32)
        idx = jnp.zeros((1, _PBE), _i32)
        for d in range(3):
            bit = (s >> d) & 1
            fd = frac[d:d + 1, :]
            w = w * (fd if bit else (1.0 - fd))
            idx = idx + (boti[d:d + 1, :] + bit) * (4 ** d)
        ws.append(w)
        gs.append(src * NK + idx)
    w_ref[...] = jnp.concatenate(ws, axis=0)
    g_ref[...] = jnp.concatenate(gs, axis=0)


def _prep(pseudoT, srcT):
    grid = E // _PBE
    return pl.pallas_call(
        _prep_body,
        grid=(grid,),
        in_specs=[
            pl.BlockSpec((3, _PBE), lambda i: (0, i)),
            pl.BlockSpec((1, _PBE), lambda i: (0, i)),
        ],
        out_specs=[
            pl.BlockSpec((8, _PBE), lambda i: (0, i)),
            pl.BlockSpec((8, _PBE), lambda i: (0, i)),
        ],
        out_shape=[
            jax.ShapeDtypeStruct((8, E), _f32),
            jax.ShapeDtypeStruct((8, E), _i32),
        ],
    )(pseudoT, srcT)


_Y1BN = 200


def _y1_body(x_ref, w_ref, o_ref):
    xb = x_ref[...]
    w = w_ref[...]
    t = xb[:, :, None] * w[None, :, :]
    col = (lax.broadcasted_iota(_i32, (1, 1, 128), 2) == NK).astype(_f32)
    o_ref[...] = (t + col).reshape(_Y1BN * NK, 128)


def _y1build(x, w1pad):
    grid = N // _Y1BN
    return pl.pallas_call(
        _y1_body,
        grid=(grid,),
        in_specs=[
            pl.BlockSpec((_Y1BN, 1), lambda i: (i, 0)),
            pl.BlockSpec((NK, 128), lambda i: (0, 0)),
        ],
        out_specs=pl.BlockSpec((_Y1BN * NK, 128), lambda i: (i, 0)),
        out_shape=jax.ShapeDtypeStruct((N * NK, 128), _f32),
    )(x, w1pad)


def _mm_body(a_ref, b_ref, o_ref):
    o_ref[...] = jnp.dot(a_ref[...], b_ref[...], preferred_element_type=_f32)


def _matmul(a, b, bm=1000, bn=512):
    m, k = a.shape
    k2, n = b.shape
    assert k == k2 and m % bm == 0
    bn = min(bn, n)
    assert n % bn == 0
    return pl.pallas_call(
        _mm_body,
        grid=(m // bm, n // bn),
        in_specs=[
            pl.BlockSpec((bm, k), lambda i, j: (i, 0)),
            pl.BlockSpec((k, bn), lambda i, j: (0, j)),
        ],
        out_specs=pl.BlockSpec((bm, bn), lambda i, j: (i, j)),
        out_shape=jax.ShapeDtypeStruct((m, n), _f32),
    )(a, b)


@functools.lru_cache(maxsize=None)
def _make_gcs(D, interpret=False):
    NCH = N // 16
    mesh = plsc.VectorSubcoreMesh(core_axis_name="c", subcore_axis_name="s")

    @functools.partial(
        pl.kernel,
        out_type=jax.ShapeDtypeStruct((NSC, N, D), _f32),
        mesh=mesh,
        scratch_types=[
            pltpu.VMEM((8 * BE,), _i32),
            pltpu.VMEM((8 * BE,), _i32),
            pltpu.VMEM((16 * BE,), _f32),
            pltpu.VMEM((16 * BE,), _f32),
            pltpu.VMEM((BE,), _i32),
            pltpu.VMEM((BE,), _i32),
            pltpu.VMEM((8 * BE, D), _f32),
            pltpu.VMEM((8 * BE, D), _f32),
            pltpu.VMEM((BE, D), _f32),
            pltpu.VMEM((16, D), _f32),
            pltpu.VMEM_SHARED((N, D), _f32),
            pltpu.SemaphoreType.DMA,
            pltpu.SemaphoreType.DMA,
            pltpu.SemaphoreType.DMA,
            pltpu.SemaphoreType.DMA,
            pltpu.SemaphoreType.DMA,
        ],
        interpret=interpret,
    )
    def gcs(tab_hbm, gidx_hbm, wgt_hbm, dst_hbm, out_hbm,
            idx_a, idx_b, w_a, w_b, dst_a, dst_b, rows_a, rows_b,
            msg_v, zero_v, acc_sh, semm_a, semm_b, semr_a, semr_b, semz):
        c = lax.axis_index("c")
        s = lax.axis_index("s")

        def zrow(r, _):
            for j in range(D // 16):
                zero_v[r, pl.ds(j * 16, 16)] = jnp.zeros((16,), _f32)
            return 0
        lax.fori_loop(0, 16, zrow, 0)
        nchunk = (N // 16 - s + NT - 1) // NT

        def zdesc(q):
            return pltpu.make_async_copy(
                zero_v, acc_sh.at[pl.ds(q * 16, 16)], semz)

        def zstart(t, _):
            zdesc(s + t * NT).start()
            return 0

        def zwait(t, _):
            zdesc(s).wait()
            return 0
        lax.fori_loop(0, nchunk, zstart, 0)
        lax.fori_loop(0, nchunk, zwait, 0)
        plsc.subcore_barrier()

        blk0 = (c * NT + s) * BPT
        SETS = ((idx_a, w_a, dst_a, rows_a, semm_a, semr_a),
                (idx_b, w_b, dst_b, rows_b, semm_b, semr_b))

        def meta_descs(b, st):
            idxb, wb, dstb, _, semm, _ = st
            m0 = (blk0 + b) * (8 * BE)
            e0 = (blk0 + b) * BE
            return [
                pltpu.make_async_copy(gidx_hbm.at[pl.ds(m0, 8 * BE)],
                                      idxb, semm),
                pltpu.make_async_copy(wgt_hbm.at[pl.ds((blk0 + b) * 16 * BE,
                                                        16 * BE)],
                                      wb, semm),
                pltpu.make_async_copy(dst_hbm.at[pl.ds(e0, BE)],
                                      dstb, semm),
            ]

        def gather_desc(st):
            idxb, _, _, rowsb, _, semr = st
            return pltpu.make_async_copy(tab_hbm.at[idxb], rowsb, semr)

        def compute_scatter(st):
            _, wb, dstb, rowsb, _, _ = st

            def edge(e, _):
                wv16 = wb[pl.ds(e * 16, 16)]
                for j in range(D // 16):
                    sl = pl.ds(j * 16, 16)
                    acc = wv16[0] * rowsb[e, sl]
                    for k in range(1, 8):
                        acc = acc + wv16[k] * rowsb[k * BE + e, sl]
                    msg_v[e, sl] = acc
                return 0
            lax.fori_loop(0, BE, edge, 0)
            pltpu.sync_copy(msg_v, acc_sh.at[dstb], add=True)

        A, B = SETS
        for dsc in meta_descs(0, A):
            dsc.start()
        for dsc in meta_descs(0, A):
            dsc.wait()
        gather_desc(A).start()
        for dsc in meta_descs(1, B):
            dsc.start()

        def body(t, _):
            bb = 2 * t + 1
            for dsc in meta_descs(bb, B):
                dsc.wait()
            gather_desc(B).start()
            gather_desc(A).wait()
            compute_scatter(A)
            ba = jnp.minimum(2 * t + 2, BPT - 1)
            for dsc in meta_descs(ba, A):
                dsc.start()
            for dsc in meta_descs(ba, A):
                dsc.wait()
            gather_desc(A).start()
            gather_desc(B).wait()
            compute_scatter(B)
            bb2 = jnp.minimum(2 * t + 3, BPT - 1)
            for dsc in meta_descs(bb2, B):
                dsc.start()
            return 0
        lax.fori_loop(0, (BPT - 1) // 2, body, 0)
        gather_desc(A).wait()
        compute_scatter(A)
        for dsc in meta_descs(BPT - 1, B):
            dsc.wait()

        plsc.subcore_barrier()

        def wdesc(q):
            return pltpu.make_async_copy(
                acc_sh.at[pl.ds(q * 16, 16)],
                out_hbm.at[c, pl.ds(q * 16, 16)], semz)

        def wstart(t, _):
            wdesc(s + t * NT).start()
            return 0

        def wwait(t, _):
            wdesc(s).wait()
            return 0
        lax.fori_loop(0, nchunk, wstart, 0)
        lax.fori_loop(0, nchunk, wwait, 0)

    return gcs


def _post_body(co, first, last, parts_ref, x_ref, root_ref, bias_ref,
               g_ref, b_ref, deg_ref, o_ref, degout_ref=None):
    psum = parts_ref[0] + parts_ref[1]
    agg = psum[:, :co]
    if first:
        deg = jnp.clip(psum[:, NK:NK + 1], 1.0, None)
        degout_ref[...] = deg
    else:
        deg = deg_ref[...]
    h = agg / deg + jnp.dot(x_ref[...], root_ref[...],
                            preferred_element_type=_f32) + bias_ref[...]
    h = jnp.where(h > 0, h, jnp.exp(h) - 1.0)
    mu = jnp.mean(h, axis=0, keepdims=True)
    var = jnp.mean((h - mu) ** 2, axis=0, keepdims=True)
    h = (h - mu) / jnp.sqrt(var + 1e-5) * g_ref[...] + b_ref[...]
    if last:
        shifted = h - jnp.max(h, axis=1, keepdims=True)
        h = shifted - jnp.log(jnp.sum(jnp.exp(shifted), axis=1, keepdims=True))
    o_ref[...] = h


def _post(parts, x, root, bias, g, b, deg, first=False, last=False):
    co = root.shape[1]
    D = parts.shape[2]
    outs = [jax.ShapeDtypeStruct((N, co), _f32)]
    if first:
        outs.append(jax.ShapeDtypeStruct((N, 1), _f32))
    res = pl.pallas_call(
        functools.partial(_post_body, co, first, last),
        out_shape=outs,
    )(parts, x, root, bias.reshape(1, co), g.reshape(1, co),
      b.reshape(1, co), deg)
    return res if first else res[0]


def kernel(x, edge_index, edge_attr, W1, root1, bias1, g1, b1,
           W2, root2, bias2, g2, b2, W3, root3, bias3, g3, b3,
           W4, root4, bias4, g4, b4):
    src = edge_index[0].astype(_i32)
    dst = edge_index[1].astype(_i32)
    pseudoT = edge_attr.T
    wgt8, gidx8 = _prep(pseudoT, src.reshape(1, E))
    pad = EP - E
    gidxp = jnp.concatenate([gidx8, jnp.zeros((8, pad), _i32)], axis=1)
    wgtp = jnp.concatenate([wgt8, jnp.zeros((8, pad), _f32)], axis=1)
    gidx8 = gidxp.reshape(8, EP // BE, BE).transpose(1, 0, 2).reshape(8 * EP)
    wgt8 = jnp.pad(wgtp.T, ((0, 0), (0, 8))).reshape(16 * EP)
    dst = jnp.concatenate([dst, jnp.zeros((pad,), _i32)])

    w1pad = jnp.pad(W1.reshape(NK, 64), ((0, 0), (0, 64)))
    tab1 = _y1build(x, w1pad)
    parts1 = _make_gcs(128)(tab1, gidx8, wgt8, dst)
    h1, deg = _post(parts1, x, root1, bias1, g1, b1, jnp.zeros((N, 1), _f32),
                    first=True)

    w2cat = W2.transpose(1, 0, 2).reshape(64, NK * 128)
    tab2 = _matmul(h1, w2cat).reshape(N * NK, 128)
    parts2 = _make_gcs(128)(tab2, gidx8, wgt8, dst)
    h2 = _post(parts2, h1, root2, bias2, g2, b2, deg)

    halves = []
    for p in range(2):
        w3p = W3[:, :, p * 128:(p + 1) * 128].transpose(1, 0, 2)
        tabp = _matmul(h2, w3p.reshape(128, NK * 128)).reshape(N * NK, 128)
        partsp = _make_gcs(128)(tabp, gidx8, wgt8, dst)
        halves.append(_post(partsp, h2, root3[:, p * 128:(p + 1) * 128],
                            bias3[p * 128:(p + 1) * 128],
                            g3[p * 128:(p + 1) * 128],
                            b3[p * 128:(p + 1) * 128], deg))
    h3 = jnp.concatenate(halves, axis=1)

    y4 = _matmul(h3, W4[:, :, 0].T)
    tab4 = jnp.pad(y4.reshape(N * NK, 1), ((0, 0), (0, 127)))
    parts4 = _make_gcs(128)(tab4, gidx8, wgt8, dst)
    out = _post(parts4, h3, root4, bias4, g4, b4, deg, last=True)
    return out


# speedup vs baseline: 2.5013x; 1.1775x over previous
import functools

import jax
import jax.numpy as jnp
from jax import lax
from jax.experimental import pallas as pl
from jax.experimental.pallas import tpu as pltpu
from jax.experimental.pallas import tpu_sc as plsc

N = 10000
E = 160000
NK = 64
NSC = 2
NT = 16
EPC = E // NSC
NPT = N // NT
BE = 16
BPT = 313
EP = NSC * NT * BPT * BE

_f32 = jnp.float32
_i32 = jnp.int32


_PBE = 1280


def _prep_body(p_ref, s_ref, w_ref, g_ref, b_ref):
    v = p_ref[...] * 3.0
    bot = jnp.clip(jnp.floor(v), 0.0, 2.0)
    frac = v - bot
    boti = bot.astype(_i32)
    src = s_ref[...]
    lane = lax.broadcasted_iota(_i32, (NK, _PBE), 0)
    ws, gs = [], []
    bd = jnp.zeros((NK, _PBE), _f32)
    for s in range(8):
        w = jnp.ones((1, _PBE), _f32)
        idx = jnp.zeros((1, _PBE), _i32)
        for d in range(3):
            bit = (s >> d) & 1
            fd = frac[d:d + 1, :]
            w = w * (fd if bit else (1.0 - fd))
            idx = idx + (boti[d:d + 1, :] + bit) * (4 ** d)
        ws.append(w)
        gs.append(src * NK + idx)
        bd = bd + jnp.where(lane == idx, w, 0.0)
    w_ref[...] = jnp.concatenate(ws, axis=0)
    g_ref[...] = jnp.concatenate(gs, axis=0)
    b_ref[...] = bd


def _prep(pseudoT, srcT):
    grid = E // _PBE
    return pl.pallas_call(
        _prep_body,
        grid=(grid,),
        in_specs=[
            pl.BlockSpec((3, _PBE), lambda i: (0, i)),
            pl.BlockSpec((1, _PBE), lambda i: (0, i)),
        ],
        out_specs=[
            pl.BlockSpec((8, _PBE), lambda i: (0, i)),
            pl.BlockSpec((8, _PBE), lambda i: (0, i)),
            pl.BlockSpec((NK, _PBE), lambda i: (0, i)),
        ],
        out_shape=[
            jax.ShapeDtypeStruct((8, E), _f32),
            jax.ShapeDtypeStruct((8, E), _i32),
            jax.ShapeDtypeStruct((NK, E), _f32),
        ],
    )(pseudoT, srcT)


_Y1BN = 200
_D1 = 128


def _y1_body(x_ref, w_ref, o_ref):
    xb = x_ref[...]
    w = w_ref[...]
    t = xb[:, :, None] * w[None, :, :]
    col = (lax.broadcasted_iota(_i32, (1, 1, _D1), 2) == NK).astype(_f32)
    o_ref[...] = (t + col).reshape(_Y1BN * NK, _D1)


def _y1build(x, w1pad):
    grid = N // _Y1BN
    return pl.pallas_call(
        _y1_body,
        grid=(grid,),
        in_specs=[
            pl.BlockSpec((_Y1BN, 1), lambda i: (i, 0)),
            pl.BlockSpec((NK, _D1), lambda i: (0, 0)),
        ],
        out_specs=pl.BlockSpec((_Y1BN * NK, _D1), lambda i: (i, 0)),
        out_shape=jax.ShapeDtypeStruct((N * NK, _D1), _f32),
    )(x, w1pad)


def _mm_body(a_ref, b_ref, o_ref):
    o_ref[...] = jnp.dot(a_ref[...], b_ref[...], preferred_element_type=_f32)


def _matmul(a, b, bm=1000, bn=512):
    m, k = a.shape
    k2, n = b.shape
    assert k == k2 and m % bm == 0
    bn = min(bn, n)
    assert n % bn == 0
    return pl.pallas_call(
        _mm_body,
        grid=(m // bm, n // bn),
        in_specs=[
            pl.BlockSpec((bm, k), lambda i, j: (i, 0)),
            pl.BlockSpec((k, bn), lambda i, j: (0, j)),
        ],
        out_specs=pl.BlockSpec((bm, bn), lambda i, j: (i, j)),
        out_shape=jax.ShapeDtypeStruct((m, n), _f32),
    )(a, b)


@functools.lru_cache(maxsize=None)
def _make_gcs(D, interpret=False):
    NCH = N // 16
    mesh = plsc.VectorSubcoreMesh(core_axis_name="c", subcore_axis_name="s")

    @functools.partial(
        pl.kernel,
        out_type=jax.ShapeDtypeStruct((NSC, N, D), _f32),
        mesh=mesh,
        scratch_types=[
            pltpu.VMEM((8 * BE,), _i32),
            pltpu.VMEM((8 * BE,), _i32),
            pltpu.VMEM((16 * BE,), _f32),
            pltpu.VMEM((16 * BE,), _f32),
            pltpu.VMEM((BE,), _i32),
            pltpu.VMEM((BE,), _i32),
            pltpu.VMEM((8 * BE, D), _f32),
            pltpu.VMEM((8 * BE, D), _f32),
            pltpu.VMEM((BE, D), _f32),
            pltpu.VMEM((16, D), _f32),
            pltpu.VMEM_SHARED((N, D), _f32),
            pltpu.SemaphoreType.DMA,
            pltpu.SemaphoreType.DMA,
            pltpu.SemaphoreType.DMA,
            pltpu.SemaphoreType.DMA,
            pltpu.SemaphoreType.DMA,
        ],
        interpret=interpret,
    )
    def gcs(tab_hbm, gidx_hbm, wgt_hbm, dst_hbm, out_hbm,
            idx_a, idx_b, w_a, w_b, dst_a, dst_b, rows_a, rows_b,
            msg_v, zero_v, acc_sh, semm_a, semm_b, semr_a, semr_b, semz):
        c = lax.axis_index("c")
        s = lax.axis_index("s")

        def zrow(r, _):
            for j in range(D // 16):
                zero_v[r, pl.ds(j * 16, 16)] = jnp.zeros((16,), _f32)
            return 0
        lax.fori_loop(0, 16, zrow, 0)
        nchunk = (N // 16 - s + NT - 1) // NT

        def zdesc(q):
            return pltpu.make_async_copy(
                zero_v, acc_sh.at[pl.ds(q * 16, 16)], semz)

        def zstart(t, _):
            zdesc(s + t * NT).start()
            return 0

        def zwait(t, _):
            zdesc(s).wait()
            return 0
        lax.fori_loop(0, nchunk, zstart, 0)
        lax.fori_loop(0, nchunk, zwait, 0)
        plsc.subcore_barrier()

        blk0 = (c * NT + s) * BPT
        SETS = ((idx_a, w_a, dst_a, rows_a, semm_a, semr_a),
                (idx_b, w_b, dst_b, rows_b, semm_b, semr_b))

        def meta_descs(b, st):
            idxb, wb, dstb, _, semm, _ = st
            m0 = (blk0 + b) * (8 * BE)
            e0 = (blk0 + b) * BE
            return [
                pltpu.make_async_copy(gidx_hbm.at[pl.ds(m0, 8 * BE)],
                                      idxb, semm),
                pltpu.make_async_copy(wgt_hbm.at[pl.ds((blk0 + b) * 16 * BE,
                                                        16 * BE)],
                                      wb, semm),
                pltpu.make_async_copy(dst_hbm.at[pl.ds(e0, BE)],
                                      dstb, semm),
            ]

        def gather_desc(st):
            idxb, _, _, rowsb, _, semr = st
            return pltpu.make_async_copy(tab_hbm.at[idxb], rowsb, semr)

        def compute_scatter(st):
            _, wb, dstb, rowsb, _, _ = st

            def edge(e, _):
                wv16 = wb[pl.ds(e * 16, 16)]
                for j in range(D // 16):
                    sl = pl.ds(j * 16, 16)
                    acc = wv16[0] * rowsb[e, sl]
                    for k in range(1, 8):
                        acc = acc + wv16[k] * rowsb[k * BE + e, sl]
                    msg_v[e, sl] = acc
                return 0
            lax.fori_loop(0, BE, edge, 0)
            pltpu.sync_copy(msg_v, acc_sh.at[dstb], add=True)

        A, B = SETS
        for dsc in meta_descs(0, A):
            dsc.start()
        for dsc in meta_descs(0, A):
            dsc.wait()
        gather_desc(A).start()
        for dsc in meta_descs(1, B):
            dsc.start()

        def body(t, _):
            bb = 2 * t + 1
            for dsc in meta_descs(bb, B):
                dsc.wait()
            gather_desc(B).start()
            gather_desc(A).wait()
            compute_scatter(A)
            ba = jnp.minimum(2 * t + 2, BPT - 1)
            for dsc in meta_descs(ba, A):
                dsc.start()
            for dsc in meta_descs(ba, A):
                dsc.wait()
            gather_desc(A).start()
            gather_desc(B).wait()
            compute_scatter(B)
            bb2 = jnp.minimum(2 * t + 3, BPT - 1)
            for dsc in meta_descs(bb2, B):
                dsc.start()
            return 0
        lax.fori_loop(0, (BPT - 1) // 2, body, 0)
        gather_desc(A).wait()
        compute_scatter(A)
        for dsc in meta_descs(BPT - 1, B):
            dsc.wait()

        plsc.subcore_barrier()

        def wdesc(q):
            return pltpu.make_async_copy(
                acc_sh.at[pl.ds(q * 16, 16)],
                out_hbm.at[c, pl.ds(q * 16, 16)], semz)

        def wstart(t, _):
            wdesc(s + t * NT).start()
            return 0

        def wwait(t, _):
            wdesc(s).wait()
            return 0
        lax.fori_loop(0, nchunk, wstart, 0)
        lax.fori_loop(0, nchunk, wwait, 0)

    return gcs


@functools.lru_cache(maxsize=None)
def _make_gcs4(interpret=False):
    D = 128
    mesh = plsc.VectorSubcoreMesh(core_axis_name="c", subcore_axis_name="s")

    @functools.partial(
        pl.kernel,
        out_type=jax.ShapeDtypeStruct((NSC, N, D), _f32),
        mesh=mesh,
        scratch_types=[
            pltpu.VMEM((BE,), _i32),
            pltpu.VMEM((BE,), _i32),
            pltpu.VMEM((NK * BE,), _f32),
            pltpu.VMEM((NK * BE,), _f32),
            pltpu.VMEM((BE,), _i32),
            pltpu.VMEM((BE,), _i32),
            pltpu.VMEM((BE, D), _f32),
            pltpu.VMEM((BE, D), _f32),
            pltpu.VMEM((BE, D), _f32),
            pltpu.VMEM((16, D), _f32),
            pltpu.VMEM_SHARED((N, D), _f32),
            pltpu.SemaphoreType.DMA,
            pltpu.SemaphoreType.DMA,
            pltpu.SemaphoreType.DMA,
            pltpu.SemaphoreType.DMA,
            pltpu.SemaphoreType.DMA,
        ],
        interpret=interpret,
    )
    def gcs4(tab_hbm, src_hbm, bden_hbm, dst_hbm, out_hbm,
             idx_a, idx_b, b_a, b_b, dst_a, dst_b, rows_a, rows_b,
             msg_v, zero_v, acc_sh, semm_a, semm_b, semr_a, semr_b, semz):
        c = lax.axis_index("c")
        s = lax.axis_index("s")

        def zrow(r, _):
            for j in range(D // 16):
                zero_v[r, pl.ds(j * 16, 16)] = jnp.zeros((16,), _f32)
            return 0
        lax.fori_loop(0, 16, zrow, 0)

        def zmsg(e, _):
            for j in range(D // 16):
                msg_v[e, pl.ds(j * 16, 16)] = jnp.zeros((16,), _f32)
            return 0
        lax.fori_loop(0, BE, zmsg, 0)
        nchunk = (N // 16 - s + NT - 1) // NT

        def zdesc(q):
            return pltpu.make_async_copy(
                zero_v, acc_sh.at[pl.ds(q * 16, 16)], semz)

        def zstart(t, _):
            zdesc(s + t * NT).start()
            return 0

        def zwait(t, _):
            zdesc(s).wait()
            return 0
        lax.fori_loop(0, nchunk, zstart, 0)
        lax.fori_loop(0, nchunk, zwait, 0)
        plsc.subcore_barrier()

        blk0 = (c * NT + s) * BPT
        SETS = ((idx_a, b_a, dst_a, rows_a, semm_a, semr_a),
                (idx_b, b_b, dst_b, rows_b, semm_b, semr_b))

        def meta_descs(b, st):
            idxb, bb, dstb, _, semm, _ = st
            e0 = (blk0 + b) * BE
            return [
                pltpu.make_async_copy(src_hbm.at[pl.ds(e0, BE)], idxb, semm),
                pltpu.make_async_copy(bden_hbm.at[pl.ds(e0 * NK, NK * BE)],
                                      bb, semm),
                pltpu.make_async_copy(dst_hbm.at[pl.ds(e0, BE)], dstb, semm),
            ]

        def gather_desc(st):
            idxb, _, _, rowsb, _, semr = st
            return pltpu.make_async_copy(tab_hbm.at[idxb], rowsb, semr)

        def compute_scatter(st):
            _, bb, dstb, rowsb, _, _ = st

            def edge(e, _):
                acc = bb[pl.ds(e * NK, 16)] * rowsb[e, pl.ds(0, 16)]
                for j in range(1, 4):
                    acc = acc + (bb[pl.ds(e * NK + j * 16, 16)]
                                 * rowsb[e, pl.ds(j * 16, 16)])
                msg_v[e, pl.ds(0, 16)] = acc
                return 0
            lax.fori_loop(0, BE, edge, 0)
            pltpu.sync_copy(msg_v, acc_sh.at[dstb], add=True)

        A, B = SETS
        for dsc in meta_descs(0, A):
            dsc.start()
        for dsc in meta_descs(0, A):
            dsc.wait()
        gather_desc(A).start()
        for dsc in meta_descs(1, B):
            dsc.start()

        def body(t, _):
            bb = 2 * t + 1
            for dsc in meta_descs(bb, B):
                dsc.wait()
            gather_desc(B).start()
            gather_desc(A).wait()
            compute_scatter(A)
            ba = jnp.minimum(2 * t + 2, BPT - 1)
            for dsc in meta_descs(ba, A):
                dsc.start()
            for dsc in meta_descs(ba, A):
                dsc.wait()
            gather_desc(A).start()
            gather_desc(B).wait()
            compute_scatter(B)
            bb2 = jnp.minimum(2 * t + 3, BPT - 1)
            for dsc in meta_descs(bb2, B):
                dsc.start()
            return 0
        lax.fori_loop(0, (BPT - 1) // 2, body, 0)
        gather_desc(A).wait()
        compute_scatter(A)
        for dsc in meta_descs(BPT - 1, B):
            dsc.wait()

        plsc.subcore_barrier()

        def wdesc(q):
            return pltpu.make_async_copy(
                acc_sh.at[pl.ds(q * 16, 16)],
                out_hbm.at[c, pl.ds(q * 16, 16)], semz)

        def wstart(t, _):
            wdesc(s + t * NT).start()
            return 0

        def wwait(t, _):
            wdesc(s).wait()
            return 0
        lax.fori_loop(0, nchunk, wstart, 0)
        lax.fori_loop(0, nchunk, wwait, 0)

    return gcs4


def _post_body(co, first, last, fold, parts_ref, x_ref, root_ref, bias_ref,
               g_ref, b_ref, deg_ref, o_ref, degout_ref=None):
    psum = parts_ref[0] + parts_ref[1]
    if fold:
        agg = jnp.sum(psum[:, :16], axis=1, keepdims=True)
    else:
        agg = psum[:, :co]
    if first:
        deg = jnp.clip(psum[:, NK:NK + 1], 1.0, None)
        degout_ref[...] = deg
    else:
        deg = deg_ref[...]
    h = agg / deg + jnp.dot(x_ref[...], root_ref[...],
                            preferred_element_type=_f32) + bias_ref[...]
    h = jnp.where(h > 0, h, jnp.exp(h) - 1.0)
    mu = jnp.mean(h, axis=0, keepdims=True)
    var = jnp.mean((h - mu) ** 2, axis=0, keepdims=True)
    h = (h - mu) / jnp.sqrt(var + 1e-5) * g_ref[...] + b_ref[...]
    if last:
        shifted = h - jnp.max(h, axis=1, keepdims=True)
        h = shifted - jnp.log(jnp.sum(jnp.exp(shifted), axis=1, keepdims=True))
    o_ref[...] = h


def _post(parts, x, root, bias, g, b, deg, first=False, last=False,
          fold=False):
    co = root.shape[1]
    D = parts.shape[2]
    outs = [jax.ShapeDtypeStruct((N, co), _f32)]
    if first:
        outs.append(jax.ShapeDtypeStruct((N, 1), _f32))
    res = pl.pallas_call(
        functools.partial(_post_body, co, first, last, fold),
        out_shape=outs,
    )(parts, x, root, bias.reshape(1, co), g.reshape(1, co),
      b.reshape(1, co), deg)
    return res if first else res[0]


def kernel(x, edge_index, edge_attr, W1, root1, bias1, g1, b1,
           W2, root2, bias2, g2, b2, W3, root3, bias3, g3, b3,
           W4, root4, bias4, g4, b4):
    src = edge_index[0].astype(_i32)
    dst = edge_index[1].astype(_i32)
    pseudoT = edge_attr.T
    wgt8, gidx8, bden = _prep(pseudoT, src.reshape(1, E))
    pad = EP - E
    gidxp = jnp.concatenate([gidx8, jnp.zeros((8, pad), _i32)], axis=1)
    wgtp = jnp.concatenate([wgt8, jnp.zeros((8, pad), _f32)], axis=1)
    gidx8 = gidxp.reshape(8, EP // BE, BE).transpose(1, 0, 2).reshape(8 * EP)
    wgt8 = jnp.pad(wgtp.T, ((0, 0), (0, 8))).reshape(16 * EP)
    dst = jnp.concatenate([dst, jnp.zeros((pad,), _i32)])
    srcpk = jnp.concatenate([src, jnp.zeros((pad,), _i32)])
    bpk = jnp.pad(bden.T, ((0, pad), (0, 0))).reshape(EP * NK)

    w1pad = jnp.pad(W1.reshape(NK, 64), ((0, 0), (0, _D1 - 64)))
    tab1 = _y1build(x, w1pad)
    parts1 = _make_gcs(_D1)(tab1, gidx8, wgt8, dst)
    h1, deg = _post(parts1, x, root1, bias1, g1, b1, jnp.zeros((N, 1), _f32),
                    first=True)

    w2cat = W2.transpose(1, 0, 2).reshape(64, NK * 128)
    tab2 = _matmul(h1, w2cat).reshape(N * NK, 128)
    parts2 = _make_gcs(128)(tab2, gidx8, wgt8, dst)
    h2 = _post(parts2, h1, root2, bias2, g2, b2, deg)

    halves = []
    for p in range(2):
        w3p = W3[:, :, p * 128:(p + 1) * 128].transpose(1, 0, 2)
        tabp = _matmul(h2, w3p.reshape(128, NK * 128)).reshape(N * NK, 128)
        partsp = _make_gcs(128)(tabp, gidx8, wgt8, dst)
        halves.append(_post(partsp, h2, root3[:, p * 128:(p + 1) * 128],
                            bias3[p * 128:(p + 1) * 128],
                            g3[p * 128:(p + 1) * 128],
                            b3[p * 128:(p + 1) * 128], deg))
    h3 = jnp.concatenate(halves, axis=1)

    y4 = _matmul(h3, W4[:, :, 0].T)
    tab4 = jnp.pad(y4, ((0, 0), (0, 64)))
    parts4 = _make_gcs4()(tab4, srcpk, bpk, dst)
    out = _post(parts4, h3, root4, bias4, g4, b4, deg, last=True, fold=True)
    return out
